# baseline (speedup 1.0000x reference)
import functools

import jax
import jax.numpy as jnp
from jax.experimental import pallas as pl
from jax.experimental.pallas import tpu as pltpu

_NUM_GRAPHS = 256


def _round_up(x, m):
    return (x + m - 1) // m * m


def _fused_kernel(batch_ref, embed_ref, w1_ref, b1_ref, w2_ref, b2_ref,
                  out_ref, sum_ref, cnt_ref):
    n_step = pl.program_id(0)
    n_steps = pl.num_programs(0)

    @pl.when(n_step == 0)
    def _init():
        sum_ref[...] = jnp.zeros_like(sum_ref)
        cnt_ref[...] = jnp.zeros_like(cnt_ref)

    ids = batch_ref[...]
    g = sum_ref.shape[0]
    tile_n = ids.shape[1]
    graph_ids = jax.lax.broadcasted_iota(jnp.int32, (g, tile_n), 0)
    mask = graph_ids == ids

    onehot = mask.astype(jnp.bfloat16)
    emb = embed_ref[...].astype(jnp.bfloat16)
    sum_ref[...] += jnp.dot(onehot, emb, preferred_element_type=jnp.float32)
    cnt_ref[...] += jnp.sum(mask.astype(jnp.float32), axis=1, keepdims=True)

    @pl.when(n_step == n_steps - 1)
    def _head():
        inv = 1.0 / jnp.maximum(cnt_ref[...], 1.0)
        pooled = sum_ref[...] * inv
        h = jnp.dot(pooled, w1_ref[...], preferred_element_type=jnp.float32)
        h = jnp.maximum(h + b1_ref[...], 0.0)
        out = jnp.dot(h, w2_ref[...], preferred_element_type=jnp.float32)
        out_ref[...] = out + b2_ref[...]


@functools.partial(jax.jit, static_argnames=("tile_n",))
def _forward(embed, batch, w1, b1, w2, b2, tile_n=2048):
    n, d = embed.shape
    c = w2.shape[1]
    g = _NUM_GRAPHS

    n_pad = _round_up(n, tile_n)
    embed_p = jnp.pad(embed, ((0, n_pad - n), (0, 0)))
    batch_p = jnp.pad(batch.astype(jnp.int32), (0, n_pad - n),
                      constant_values=-1).reshape(1, n_pad)
    n_tiles = n_pad // tile_n

    out = pl.pallas_call(
        _fused_kernel,
        out_shape=jax.ShapeDtypeStruct((g, c), jnp.float32),
        grid=(n_tiles,),
        in_specs=[
            pl.BlockSpec((1, tile_n), lambda i: (0, i)),
            pl.BlockSpec((tile_n, d), lambda i: (i, 0)),
            pl.BlockSpec((d, d), lambda i: (0, 0)),
            pl.BlockSpec((1, d), lambda i: (0, 0)),
            pl.BlockSpec((d, c), lambda i: (0, 0)),
            pl.BlockSpec((1, c), lambda i: (0, 0)),
        ],
        out_specs=pl.BlockSpec((g, c), lambda i: (0, 0)),
        scratch_shapes=[
            pltpu.VMEM((g, d), jnp.float32),
            pltpu.VMEM((g, 1), jnp.float32),
        ],
        compiler_params=pltpu.CompilerParams(
            dimension_semantics=("arbitrary",),
            vmem_limit_bytes=64 * 1024 * 1024,
        ),
    )(batch_p, embed_p, w1, b1.reshape(1, d), w2, b2.reshape(1, c))

    return out


def kernel(embed, batch, w1, b1, w2, b2):
    return _forward(embed, batch, w1, b1, w2, b2)


# speedup vs baseline: 1.0756x; 1.0756x over previous
import functools

import jax
import jax.numpy as jnp
from jax.experimental import pallas as pl
from jax.experimental.pallas import tpu as pltpu

_NUM_GRAPHS = 256


def _round_up(x, m):
    return (x + m - 1) // m * m


def _fused_kernel(batch_ref, embed_ref, w1_ref, b1_ref, w2_ref, b2_ref,
                  out_ref, sum_ref, cnt_ref):
    n_step = pl.program_id(0)
    n_steps = pl.num_programs(0)

    @pl.when(n_step == 0)
    def _init():
        sum_ref[...] = jnp.zeros_like(sum_ref)
        cnt_ref[...] = jnp.zeros_like(cnt_ref)

    ids = batch_ref[...]
    g = sum_ref.shape[0]
    tile_n = ids.shape[1]
    graph_ids = jax.lax.broadcasted_iota(jnp.int32, (g, tile_n), 0)
    mask = graph_ids == ids

    onehot = mask.astype(jnp.bfloat16)
    emb = embed_ref[...].astype(jnp.bfloat16)
    sum_ref[...] += jnp.dot(onehot, emb, preferred_element_type=jnp.float32)
    cnt_ref[...] += jnp.sum(mask.astype(jnp.float32), axis=1, keepdims=True)

    @pl.when(n_step == n_steps - 1)
    def _head():
        inv = 1.0 / jnp.maximum(cnt_ref[...], 1.0)
        pooled = sum_ref[...] * inv
        h = jnp.dot(pooled, w1_ref[...], preferred_element_type=jnp.float32)
        h = jnp.maximum(h + b1_ref[...], 0.0)
        out = jnp.dot(h, w2_ref[...], preferred_element_type=jnp.float32)
        out_ref[...] = out + b2_ref[...]


@functools.partial(jax.jit, static_argnames=("tile_n",))
def _forward(embed, batch, w1, b1, w2, b2, tile_n=8192):
    n, d = embed.shape
    c = w2.shape[1]
    g = _NUM_GRAPHS

    n_pad = _round_up(n, tile_n)
    embed_p = jnp.pad(embed, ((0, n_pad - n), (0, 0)))
    batch_p = jnp.pad(batch.astype(jnp.int32), (0, n_pad - n),
                      constant_values=-1).reshape(1, n_pad)
    n_tiles = n_pad // tile_n

    out = pl.pallas_call(
        _fused_kernel,
        out_shape=jax.ShapeDtypeStruct((g, c), jnp.float32),
        grid=(n_tiles,),
        in_specs=[
            pl.BlockSpec((1, tile_n), lambda i: (0, i)),
            pl.BlockSpec((tile_n, d), lambda i: (i, 0)),
            pl.BlockSpec((d, d), lambda i: (0, 0)),
            pl.BlockSpec((1, d), lambda i: (0, 0)),
            pl.BlockSpec((d, c), lambda i: (0, 0)),
            pl.BlockSpec((1, c), lambda i: (0, 0)),
        ],
        out_specs=pl.BlockSpec((g, c), lambda i: (0, 0)),
        scratch_shapes=[
            pltpu.VMEM((g, d), jnp.float32),
            pltpu.VMEM((g, 1), jnp.float32),
        ],
        compiler_params=pltpu.CompilerParams(
            dimension_semantics=("arbitrary",),
            vmem_limit_bytes=64 * 1024 * 1024,
        ),
    )(batch_p, embed_p, w1, b1.reshape(1, d), w2, b2.reshape(1, c))

    return out


def kernel(embed, batch, w1, b1, w2, b2):
    return _forward(embed, batch, w1, b1, w2, b2)


# speedup vs baseline: 1.1503x; 1.0694x over previous
import functools

import jax
import jax.numpy as jnp
from jax.experimental import pallas as pl
from jax.experimental.pallas import tpu as pltpu

_NUM_GRAPHS = 256


def _round_up(x, m):
    return (x + m - 1) // m * m


def _fused_kernel(batch_ref, embed_ref, w1_ref, b1_ref, w2_ref, b2_ref,
                  out_ref, sum_ref, cnt_ref):
    n_step = pl.program_id(0)
    n_steps = pl.num_programs(0)

    @pl.when(n_step == 0)
    def _init():
        sum_ref[...] = jnp.zeros_like(sum_ref)
        cnt_ref[...] = jnp.zeros_like(cnt_ref)

    ids = batch_ref[...]
    g = sum_ref.shape[0]
    tile_n = ids.shape[1]
    graph_ids = jax.lax.broadcasted_iota(jnp.int32, (g, tile_n), 0)
    mask = graph_ids == ids

    onehot = mask.astype(jnp.bfloat16)
    emb = embed_ref[...].astype(jnp.bfloat16)
    sum_ref[...] += jnp.dot(onehot, emb, preferred_element_type=jnp.float32)
    cnt_ref[...] += jnp.sum(mask.astype(jnp.float32), axis=1, keepdims=True)

    @pl.when(n_step == n_steps - 1)
    def _head():
        inv = 1.0 / jnp.maximum(cnt_ref[...], 1.0)
        pooled = sum_ref[...] * inv
        h = jnp.dot(pooled, w1_ref[...], preferred_element_type=jnp.float32)
        h = jnp.maximum(h + b1_ref[...], 0.0)
        out = jnp.dot(h, w2_ref[...], preferred_element_type=jnp.float32)
        out_ref[...] = out + b2_ref[...]


@functools.partial(jax.jit, static_argnames=("tile_n",))
def _forward(embed, batch, w1, b1, w2, b2, tile_n=4096):
    n, d = embed.shape
    c = w2.shape[1]
    g = _NUM_GRAPHS

    n_pad = _round_up(n, tile_n)
    embed_p = jnp.pad(embed, ((0, n_pad - n), (0, 0)))
    batch_p = jnp.pad(batch.astype(jnp.int32), (0, n_pad - n),
                      constant_values=-1).reshape(1, n_pad)
    n_tiles = n_pad // tile_n

    out = pl.pallas_call(
        _fused_kernel,
        out_shape=jax.ShapeDtypeStruct((g, c), jnp.float32),
        grid=(n_tiles,),
        in_specs=[
            pl.BlockSpec((1, tile_n), lambda i: (0, i)),
            pl.BlockSpec((tile_n, d), lambda i: (i, 0)),
            pl.BlockSpec((d, d), lambda i: (0, 0)),
            pl.BlockSpec((1, d), lambda i: (0, 0)),
            pl.BlockSpec((d, c), lambda i: (0, 0)),
            pl.BlockSpec((1, c), lambda i: (0, 0)),
        ],
        out_specs=pl.BlockSpec((g, c), lambda i: (0, 0)),
        scratch_shapes=[
            pltpu.VMEM((g, d), jnp.float32),
            pltpu.VMEM((g, 1), jnp.float32),
        ],
        compiler_params=pltpu.CompilerParams(
            dimension_semantics=("arbitrary",),
            vmem_limit_bytes=64 * 1024 * 1024,
        ),
    )(batch_p, embed_p, w1, b1.reshape(1, d), w2, b2.reshape(1, c))

    return out


def kernel(embed, batch, w1, b1, w2, b2):
    return _forward(embed, batch, w1, b1, w2, b2)
